# baseline (speedup 1.0000x reference)
import functools
from itertools import combinations

import numpy as np
import jax
import jax.numpy as jnp
from jax import lax
from jax.experimental import pallas as pl
from jax.experimental.pallas import tpu as pltpu
from jax.experimental.pallas import tpu_sc as plsc

_N_WIRES = 16
_N_SHOTS = 1024.0
_N_STATES = 1 << _N_WIRES
_IDX = [(1 << (_N_WIRES - 1 - a)) + (1 << (_N_WIRES - 1 - b))
        for a, b in combinations(range(_N_WIRES), 2)]
_NB = len(_IDX)


_TC_BR = 32
_SC_ROWS = 192
_TC_ROWS = 512 - _SC_ROWS


def _tc_body(x_ref, o_ref):
    x = x_ref[...]
    s = jnp.sum(x, axis=1)
    scale = _N_SHOTS / s
    cols = [x[:, c] for c in _IDX]
    g = jnp.stack(cols, axis=1)
    o_ref[...] = g * scale[:, None]


def _tc_call(state):
    return pl.pallas_call(
        _tc_body,
        grid=(_TC_ROWS // _TC_BR,),
        in_specs=[pl.BlockSpec((_TC_BR, _N_STATES), lambda i: (i, 0))],
        out_specs=pl.BlockSpec((_TC_BR, _NB), lambda i: (i, 0)),
        out_shape=jax.ShapeDtypeStruct((_TC_ROWS, _NB), jnp.float32),
    )(state)


_CW = _N_STATES // 2
_NW = 32
_RPW = _SC_ROWS // _NW


def _make_slot_tables():
    lower = [(j, s) for j, s in enumerate(_IDX) if s < _CW]
    upper = [(j, s) for j, s in enumerate(_IDX) if s >= _CW]
    src = np.zeros(128, np.int32)
    dst = np.zeros(128, np.int32)
    for slot, (j, s) in enumerate(lower):
        src[slot], dst[slot] = s, j
    for p, slot in enumerate(range(len(lower), 112)):
        src[slot], dst[slot] = 0, 121 + p
    for slot, (j, s) in enumerate(upper, start=112):
        src[slot], dst[slot] = s - _CW, j
    src[127], dst[127] = 0, 127
    return src, dst


_TAB = np.concatenate(_make_slot_tables())
_CHUNK_OF_VEC = [0] * 7 + [1]


_NRING = 3


def _make_sc_call(base_row, n_rows, num_cores, name):
    n_workers = num_cores * 16
    rpw = n_rows // n_workers

    def _sc_body(state_hbm, tab_hbm, out_ref,
                 buf0, buf1, tabv, raw, outv, sem0, sem1):
        if num_cores == 1:
            wid = lax.axis_index("s")
        else:
            wid = lax.axis_index("s") * num_cores + lax.axis_index("c")
        base = base_row + wid * rpw
        pltpu.sync_copy(tab_hbm, tabv)

        def chunk_sum(buf):
            def body(i, a):
                b = [buf[pl.ds(i * 256 + t * 16, 16)] for t in range(16)]
                a = tuple(a[t] + b[t] for t in range(8))
                return tuple(a[t] + b[8 + t] for t in range(8))
            acc = lax.fori_loop(
                0, _CW // 256, body, (jnp.zeros((16,), jnp.float32),) * 8)
            return ((acc[0] + acc[1]) + (acc[2] + acc[3])) + (
                (acc[4] + acc[5]) + (acc[6] + acc[7]))

        pltpu.async_copy(state_hbm.at[base, pl.ds(0, _CW)], buf0, sem0)

        def row_body(r, carry):
            pltpu.make_async_copy(
                state_hbm.at[base + r, pl.ds(0, _CW)], buf0, sem0).wait()
            pltpu.async_copy(
                state_hbm.at[base + r, pl.ds(_CW, _CW)], buf1, sem1)
            a0 = chunk_sum(buf0)
            for k in range(7):
                sv = tabv[pl.ds(k * 16, 16)]
                raw[pl.ds(k * 16, 16)] = plsc.load_gather(buf0, [sv])

            @pl.when(r + 1 < rpw)
            def _():
                pltpu.async_copy(
                    state_hbm.at[base + r + 1, pl.ds(0, _CW)], buf0, sem0)

            pltpu.make_async_copy(
                state_hbm.at[base + r, pl.ds(_CW, _CW)], buf1, sem1).wait()
            a1 = chunk_sum(buf1)
            sv = tabv[pl.ds(112, 16)]
            raw[pl.ds(112, 16)] = plsc.load_gather(buf1, [sv])
            total = jnp.sum(a0 + a1)
            scale = jnp.full((16,), _N_SHOTS, jnp.float32) / (
                jnp.ones((16,), jnp.float32) * total)
            for k in range(8):
                dv = tabv[pl.ds(128 + k * 16, 16)]
                plsc.store_scatter(outv, [dv], raw[pl.ds(k * 16, 16)] * scale)
            pltpu.sync_copy(outv, out_ref.at[base - base_row + r])
            return carry

        lax.fori_loop(0, rpw, row_body, 0)

    return functools.partial(
        pl.kernel,
        out_type=jax.ShapeDtypeStruct((n_rows, 128), jnp.float32),
        mesh=plsc.VectorSubcoreMesh(core_axis_name="c", subcore_axis_name="s",
                                    num_cores=num_cores, num_subcores=16),
        compiler_params=pltpu.CompilerParams(needs_layout_passes=False),
        name=name,
        scratch_types=[
            pltpu.VMEM((_CW,), jnp.float32),
            pltpu.VMEM((_CW,), jnp.float32),
            pltpu.VMEM((256,), jnp.int32),
            pltpu.VMEM((128,), jnp.float32),
            pltpu.VMEM((128,), jnp.float32),
            pltpu.SemaphoreType.DMA,
            pltpu.SemaphoreType.DMA,
        ],
    )(_sc_body)


_sc_call = _make_sc_call(_TC_ROWS, _SC_ROWS, 2, "sc_rows")


def kernel(state):
    tc_out = _tc_call(state)
    sc_out = _sc_call(state, jnp.asarray(_TAB))
    return jnp.concatenate([tc_out, sc_out[:, :_NB]], axis=0)


# speedup vs baseline: 1.4146x; 1.4146x over previous
import jax
import jax.numpy as jnp
from itertools import combinations
from jax.experimental import pallas as pl

_N_WIRES = 16
_N_SHOTS = 1024.0
_IDX = [(1 << (_N_WIRES - 1 - a)) + (1 << (_N_WIRES - 1 - b))
        for a, b in combinations(range(_N_WIRES), 2)]
_NB = len(_IDX)
_BR = 32


def _body(x_ref, o_ref):
    x = x_ref[...]
    s = jnp.sum(x, axis=1)
    scale = _N_SHOTS / s
    cols = [x[:, c] for c in _IDX]
    g = jnp.stack(cols, axis=1)
    o_ref[...] = g * scale[:, None]


def kernel(state):
    B, N = state.shape
    return pl.pallas_call(
        _body,
        grid=(B // _BR,),
        in_specs=[pl.BlockSpec((_BR, N), lambda i: (i, 0))],
        out_specs=pl.BlockSpec((_BR, _NB), lambda i: (i, 0)),
        out_shape=jax.ShapeDtypeStruct((B, _NB), jnp.float32),
    )(state)
